# Initial kernel scaffold; baseline (speedup 1.0000x reference)
import jax
import jax.numpy as jnp
from jax.experimental import pallas as pl


def kernel(x, table):
    raise NotImplementedError("write your pallas kernel here")


# speedup vs baseline: 6.3418x; 6.3418x over previous
import functools

import jax
import jax.numpy as jnp
from jax import lax
from jax.experimental import pallas as pl
from jax.experimental.pallas import tpu as pltpu
from jax.experimental.pallas import tpu_sc as plsc

D = 128
NC = 2
NS = 16
NW = NC * NS
CHUNK = 128


def _build_sc_gather(n_chunks):
    B = NW * n_chunks * CHUNK
    mesh = plsc.VectorSubcoreMesh(
        core_axis_name="c", subcore_axis_name="s",
        num_cores=NC, num_subcores=NS)

    @functools.partial(
        pl.kernel,
        out_type=jax.ShapeDtypeStruct((B, D), jnp.float32),
        mesh=mesh,
        scratch_types=[
            pltpu.VMEM((n_chunks, CHUNK), jnp.int32),
            pltpu.VMEM((CHUNK, D), jnp.float32),
            pltpu.SemaphoreType.DMA,
        ],
    )
    def sc_gather(idx_hbm, table_hbm, out_hbm, idx_v, rows_v, sem):
        wid = lax.axis_index("s") * NC + lax.axis_index("c")
        pltpu.sync_copy(idx_hbm.at[wid], idx_v)
        base = wid * (n_chunks * CHUNK)

        def chunk_body(j, carry):
            pltpu.async_copy(table_hbm.at[idx_v.at[j]], rows_v, sem).wait()
            pltpu.sync_copy(rows_v, out_hbm.at[pl.ds(base + j * CHUNK, CHUNK)])
            return carry

        lax.fori_loop(0, n_chunks, chunk_body, 0)

    return sc_gather


def kernel(x, table):
    xs, seq = x.shape
    B = xs * seq
    n_chunks = B // (NW * CHUNK)
    idx3 = x.astype(jnp.int32).reshape(NW, n_chunks, CHUNK)
    out = _build_sc_gather(n_chunks)(idx3, table)
    return out.reshape(xs, seq, D)


# speedup vs baseline: 9.1849x; 1.4483x over previous
import functools

import jax
import jax.numpy as jnp
from jax import lax
from jax.experimental import pallas as pl
from jax.experimental.pallas import tpu as pltpu
from jax.experimental.pallas import tpu_sc as plsc

D = 128
NC = 2
NS = 16
NW = NC * NS
CHUNK = 128
NBUF = 4
LOOK = 2


def _build_sc_gather(n_chunks):
    B = NW * n_chunks * CHUNK
    mesh = plsc.VectorSubcoreMesh(
        core_axis_name="c", subcore_axis_name="s",
        num_cores=NC, num_subcores=NS)

    @functools.partial(
        pl.kernel,
        out_type=jax.ShapeDtypeStruct((B, D), jnp.float32),
        mesh=mesh,
        scratch_types=[
            pltpu.VMEM((n_chunks, CHUNK), jnp.int32),
            pltpu.VMEM((NBUF, CHUNK, D), jnp.float32),
            pltpu.SemaphoreType.DMA((NBUF,)),
            pltpu.SemaphoreType.DMA((NBUF,)),
        ],
    )
    def sc_gather(idx_hbm, table_hbm, out_hbm, idx_v, rows_v, gsem, osem):
        wid = lax.axis_index("s") * NC + lax.axis_index("c")
        pltpu.sync_copy(idx_hbm.at[wid], idx_v)
        base = wid * (n_chunks * CHUNK)

        def gather(j, slot):
            return pltpu.make_async_copy(
                table_hbm.at[idx_v.at[j]], rows_v.at[slot], gsem.at[slot])

        def writeback(j, slot):
            return pltpu.make_async_copy(
                rows_v.at[slot], out_hbm.at[pl.ds(base + j * CHUNK, CHUNK)],
                osem.at[slot])

        for b in range(LOOK):
            gather(b, b).start()

        def step(j, b):
            s2 = (b + LOOK) % NBUF

            @pl.when(j + LOOK < n_chunks)
            def _():
                @pl.when(j + LOOK >= NBUF)
                def _():
                    writeback(j + LOOK - NBUF, s2).wait()
                gather(j + LOOK, s2).start()

            gather(j, b).wait()
            writeback(j, b).start()

        def outer(i, carry):
            j0 = i * NBUF
            for b in range(NBUF):
                step(j0 + b, b)
            return carry

        lax.fori_loop(0, n_chunks // NBUF, outer, 0)

        for b in range(NBUF):
            writeback(n_chunks - NBUF + b, b).wait()

    return sc_gather


def kernel(x, table):
    xs, seq = x.shape
    B = xs * seq
    n_chunks = B // (NW * CHUNK)
    idx3 = x.astype(jnp.int32).reshape(NW, n_chunks, CHUNK)
    out = _build_sc_gather(n_chunks)(idx3, table)
    return out.reshape(xs, seq, D)


# speedup vs baseline: 9.2025x; 1.0019x over previous
import functools

import jax
import jax.numpy as jnp
from jax import lax
from jax.experimental import pallas as pl
from jax.experimental.pallas import tpu as pltpu
from jax.experimental.pallas import tpu_sc as plsc

D = 128
NC = 2
NS = 16
NW = NC * NS
CHUNK = 128
NBUF = 5
LOOK = 2


def _build_sc_gather(n_chunks):
    B = NW * n_chunks * CHUNK
    mesh = plsc.VectorSubcoreMesh(
        core_axis_name="c", subcore_axis_name="s",
        num_cores=NC, num_subcores=NS)

    @functools.partial(
        pl.kernel,
        out_type=jax.ShapeDtypeStruct((B, D), jnp.float32),
        mesh=mesh,
        scratch_types=[
            pltpu.VMEM((n_chunks, CHUNK), jnp.int32),
            pltpu.VMEM((NBUF, CHUNK, D), jnp.float32),
            pltpu.SemaphoreType.DMA((NBUF,)),
            pltpu.SemaphoreType.DMA((NBUF,)),
        ],
    )
    def sc_gather(idx_hbm, table_hbm, out_hbm, idx_v, rows_v, gsem, osem):
        wid = lax.axis_index("s") * NC + lax.axis_index("c")
        pltpu.sync_copy(idx_hbm.at[wid], idx_v)
        base = wid * (n_chunks * CHUNK)

        def gather(j, slot):
            return pltpu.make_async_copy(
                table_hbm.at[idx_v.at[j]], rows_v.at[slot], gsem.at[slot])

        def writeback(j, slot):
            return pltpu.make_async_copy(
                rows_v.at[slot], out_hbm.at[pl.ds(base + j * CHUNK, CHUNK)],
                osem.at[slot])

        for b in range(LOOK):
            gather(b, b).start()

        def step(j, b):
            s2 = (b + LOOK) % NBUF

            @pl.when(j + LOOK < n_chunks)
            def _():
                @pl.when(j + LOOK >= NBUF)
                def _():
                    writeback(j + LOOK - NBUF, s2).wait()
                gather(j + LOOK, s2).start()

            gather(j, b).wait()
            writeback(j, b).start()

        def outer(i, carry):
            j0 = i * NBUF
            for b in range(NBUF):
                step(j0 + b, b)
            return carry

        lax.fori_loop(0, n_chunks // NBUF, outer, 0)

        for b in range(NBUF):
            writeback(n_chunks - NBUF + b, b).wait()

    return sc_gather


def kernel(x, table):
    xs, seq = x.shape
    B = xs * seq
    n_chunks = B // (NW * CHUNK)
    idx3 = x.astype(jnp.int32).reshape(NW, n_chunks, CHUNK)
    out = _build_sc_gather(n_chunks)(idx3, table)
    return out.reshape(xs, seq, D)


# speedup vs baseline: 9.2646x; 1.0068x over previous
import functools

import jax
import jax.numpy as jnp
from jax import lax
from jax.experimental import pallas as pl
from jax.experimental.pallas import tpu as pltpu
from jax.experimental.pallas import tpu_sc as plsc

D = 128
NC = 2
NS = 16
NW = NC * NS
CHUNK = 128
KPER = 2
NBUF = 2
LOOK = 1


def _build_sc_gather(n_slots):
    mesh = plsc.VectorSubcoreMesh(
        core_axis_name="c", subcore_axis_name="s",
        num_cores=NC, num_subcores=NS)

    @functools.partial(
        pl.kernel,
        out_type=jax.ShapeDtypeStruct((NW * n_slots, CHUNK, D),
                                      jnp.float32),
        mesh=mesh,
        scratch_types=[
            pltpu.VMEM((n_slots, CHUNK), jnp.int32),
            pltpu.VMEM((NBUF, CHUNK, D), jnp.float32),
            pltpu.SemaphoreType.DMA((NBUF,)),
            pltpu.SemaphoreType.DMA((NBUF,)),
        ],
    )
    def sc_gather(idx_hbm, table_hbm, out_hbm, idx_v, rows_v, gsem, osem):
        wid = lax.axis_index("s") * NC + lax.axis_index("c")
        pltpu.sync_copy(idx_hbm.at[wid], idx_v)
        base = wid * n_slots

        def gather(j, slot):
            return pltpu.make_async_copy(
                table_hbm.at[idx_v.at[j]], rows_v.at[slot], gsem.at[slot])

        def writeback(j, slot):
            return pltpu.make_async_copy(
                rows_v.at[slot], out_hbm.at[base + j], osem.at[slot])

        for b in range(LOOK):
            gather(b, b).start()

        def step(j, b):
            s2 = (b + LOOK) % NBUF

            @pl.when(j + LOOK < n_slots)
            def _():
                @pl.when(j + LOOK >= NBUF)
                def _():
                    writeback(j + LOOK - NBUF, s2).wait()
                gather(j + LOOK, s2).start()

            gather(j, b).wait()
            writeback(j, b).start()

        def outer(i, carry):
            j0 = i * NBUF
            for b in range(NBUF):
                step(j0 + b, b)
            return carry

        lax.fori_loop(0, n_slots // NBUF, outer, 0)

        for b in range(NBUF):
            writeback(n_slots - NBUF + b, b).wait()

    return sc_gather


def kernel(x, table):
    xs, seq = x.shape
    B = xs * seq
    n_slots = B // (NW * CHUNK)
    idx4 = x.astype(jnp.int32).reshape(NW, n_slots, CHUNK)
    out = _build_sc_gather(n_slots)(idx4, table)
    return out.reshape(xs, seq, D)
